# baseline (speedup 1.0000x reference)
import functools

import jax
import jax.numpy as jnp
from jax.experimental import pallas as pl
from jax.experimental.pallas import tpu as pltpu

_SLOTS = 4


def _pick_br(n: int) -> int:
    for br in (400, 200, 100, 40, 8):
        if n % br == 0:
            return br
    return n


def _chunk_copy(adj_ref, stage_ref, last_ref, sem_ref, j, br, f, w, bc, nbc):
    if f == nbc - 1 and w != bc:
        dst = last_ref
        sem = sem_ref.at[_SLOTS]
    else:
        dst = stage_ref.at[f % _SLOTS]
        sem = sem_ref.at[f % _SLOTS]
    return pltpu.make_async_copy(
        adj_ref.at[pl.ds(j * br, br), pl.ds(f * bc, w)], dst, sem)


def _gcn_kernel(n, br, nbr, bc, nbc, h_dim,
                x_ref, adjs_ref, adjh_ref, w1_ref, b1_ref, w2_ref, b2_ref,
                ls_ref, sm_ref, s2cat_ref, s2row_ref, lacc_ref, accb_ref,
                stage_ref, last_ref, sem_ref):
    t = pl.program_id(0)
    widths = [bc] * (nbc - 1) + [n - (nbc - 1) * bc]

    def chunk_copy(j, f):
        return _chunk_copy(adjh_ref, stage_ref, last_ref, sem_ref,
                           j, br, f, widths[f], bc, nbc)

    @pl.when(t == 0)
    def _():
        s2cat_ref[:, pl.ds(0, h_dim)] = jnp.dot(
            x_ref[...], w1_ref[...],
            preferred_element_type=jnp.float32).astype(jnp.bfloat16)
        s2cat_ref[:, pl.ds(h_dim, s2cat_ref.shape[1] - h_dim)] = jnp.zeros(
            (n, s2cat_ref.shape[1] - h_dim), jnp.bfloat16)

    @pl.when(t < nbr)
    def _():
        i = t
        big = jnp.dot(adjs_ref[...].astype(jnp.bfloat16), s2cat_ref[...],
                      preferred_element_type=jnp.float32)
        h = jnp.maximum(big[:, :h_dim] + b1_ref[...], 0.0)
        lacc_ref[pl.ds(i * br, br), :] = big[:, h_dim:]
        s2row_ref[pl.ds(i * br, br), :] = jnp.dot(
            h.astype(jnp.bfloat16), w2_ref[...].astype(jnp.bfloat16),
            preferred_element_type=jnp.float32).astype(jnp.bfloat16)

        pc = (i * br) // bc
        cc = ((i + 1) * br) // bc

        @pl.when(cc > pc)
        def _():
            s2cat_ref[pl.ds(pc * bc, bc), pl.ds(h_dim, s2row_ref.shape[1])] = (
                s2row_ref[pl.ds(pc * bc, bc), :])

    @pl.when(t >= nbr)
    def _():
        j = t - nbr
        fj = (j * br) // bc

        accb_ref[...] = lacc_ref[pl.ds(j * br, br), :] + b2_ref[...]

        for f in range(nbc):
            @pl.when(jnp.logical_and(f >= fj, f < fj + _SLOTS))
            def _(f=f):
                chunk_copy(j, f).start()

        for f in range(nbc):
            @pl.when(f >= fj)
            def _(f=f):
                chunk_copy(j, f).wait()
                if f == nbc - 1 and widths[f] != bc:
                    src = last_ref[...]
                else:
                    src = stage_ref[f % _SLOTS]
                accb_ref[...] += jnp.dot(
                    src.astype(jnp.bfloat16),
                    s2row_ref[pl.ds(f * bc, widths[f]), :],
                    preferred_element_type=jnp.float32)
                if f + _SLOTS < nbc:
                    @pl.when(f + _SLOTS >= fj)
                    def _(f=f):
                        chunk_copy(j, f + _SLOTS).start()

        logits = accb_ref[...]
        m = jnp.max(logits, axis=1, keepdims=True)
        z = logits - m
        e = jnp.exp(z)
        s = jnp.sum(e, axis=1, keepdims=True)
        sm_ref[...] = e / s
        ls_ref[...] = z - jnp.log(s)


def kernel(x, adj, W1, b1, W2, b2):
    n, f_in = x.shape
    h_dim = W1.shape[1]
    c_dim = W2.shape[1]
    br = _pick_br(n)
    nbr = n // br
    bc = 1024 if n >= 4096 else 256
    nbc = -(-n // bc)

    b1r = b1.reshape(1, h_dim)
    b2r = b2.reshape(1, c_dim)

    ls, sm = pl.pallas_call(
        functools.partial(_gcn_kernel, n, br, nbr, bc, nbc, h_dim),
        grid=(2 * nbr,),
        in_specs=[
            pl.BlockSpec((n, f_in), lambda t: (0, 0)),
            pl.BlockSpec((br, n), lambda t: (jnp.minimum(t, nbr - 1), 0)),
            pl.BlockSpec(memory_space=pl.ANY),
            pl.BlockSpec((f_in, h_dim), lambda t: (0, 0)),
            pl.BlockSpec((1, h_dim), lambda t: (0, 0)),
            pl.BlockSpec((h_dim, c_dim), lambda t: (0, 0)),
            pl.BlockSpec((1, c_dim), lambda t: (0, 0)),
        ],
        out_specs=[
            pl.BlockSpec((br, c_dim), lambda t: (jnp.maximum(t - nbr, 0), 0)),
            pl.BlockSpec((br, c_dim), lambda t: (jnp.maximum(t - nbr, 0), 0)),
        ],
        out_shape=[
            jax.ShapeDtypeStruct((n, c_dim), jnp.float32),
            jax.ShapeDtypeStruct((n, c_dim), jnp.float32),
        ],
        scratch_shapes=[
            pltpu.VMEM((n, h_dim + c_dim), jnp.bfloat16),
            pltpu.VMEM((n, c_dim), jnp.bfloat16),
            pltpu.VMEM((n, c_dim), jnp.float32),
            pltpu.VMEM((br, c_dim), jnp.float32),
            pltpu.VMEM((_SLOTS, br, bc), jnp.float32),
            pltpu.VMEM((br, n - (nbc - 1) * bc), jnp.float32),
            pltpu.SemaphoreType.DMA((_SLOTS + 1,)),
        ],
    )(x, adj, adj, W1, b1r, W2, b2r)
    return ls, sm


# speedup vs baseline: 1.0344x; 1.0344x over previous
import functools

import jax
import jax.numpy as jnp
from jax.experimental import pallas as pl
from jax.experimental.pallas import tpu as pltpu

_SLOTS = 5


def _pick_br(n: int) -> int:
    for br in (400, 200, 100, 40, 8):
        if n % br == 0:
            return br
    return n


def _chunk_copy(adj_ref, stage_ref, last_ref, sem_ref, j, br, f, w, bc, nbc):
    if f == nbc - 1 and w != bc:
        dst = last_ref
        sem = sem_ref.at[_SLOTS]
    else:
        dst = stage_ref.at[f % _SLOTS]
        sem = sem_ref.at[f % _SLOTS]
    return pltpu.make_async_copy(
        adj_ref.at[pl.ds(j * br, br), pl.ds(f * bc, w)], dst, sem)


def _gcn_kernel(n, br, nbr, bc, nbc, h_dim,
                x_ref, adjs_ref, adjh_ref, w1_ref, b1_ref, w2_ref, b2_ref,
                ls_ref, sm_ref, s2cat_ref, s2row_ref, lacc_ref, accb_ref,
                stage_ref, last_ref, sem_ref):
    t = pl.program_id(0)
    widths = [bc] * (nbc - 1) + [n - (nbc - 1) * bc]

    def chunk_copy(j, f):
        return _chunk_copy(adjh_ref, stage_ref, last_ref, sem_ref,
                           j, br, f, widths[f], bc, nbc)

    @pl.when(t == 0)
    def _():
        s2cat_ref[:, pl.ds(0, h_dim)] = jnp.dot(
            x_ref[...], w1_ref[...],
            preferred_element_type=jnp.float32).astype(jnp.bfloat16)
        s2cat_ref[:, pl.ds(h_dim, s2cat_ref.shape[1] - h_dim)] = jnp.zeros(
            (n, s2cat_ref.shape[1] - h_dim), jnp.bfloat16)

    @pl.when(t < nbr)
    def _():
        i = t
        big = jnp.dot(adjs_ref[...].astype(jnp.bfloat16), s2cat_ref[...],
                      preferred_element_type=jnp.float32)
        h = jnp.maximum(big[:, :h_dim] + b1_ref[...], 0.0)
        lacc_ref[pl.ds(i * br, br), :] = big[:, h_dim:]
        s2row_ref[pl.ds(i * br, br), :] = jnp.dot(
            h.astype(jnp.bfloat16), w2_ref[...].astype(jnp.bfloat16),
            preferred_element_type=jnp.float32).astype(jnp.bfloat16)

        pc = (i * br) // bc
        cc = ((i + 1) * br) // bc

        @pl.when(cc > pc)
        def _():
            s2cat_ref[pl.ds(pc * bc, bc), pl.ds(h_dim, s2row_ref.shape[1])] = (
                s2row_ref[pl.ds(pc * bc, bc), :])

    @pl.when(t >= nbr)
    def _():
        j = t - nbr
        fj = (j * br) // bc

        accb_ref[...] = lacc_ref[pl.ds(j * br, br), :] + b2_ref[...]

        @pl.when(t == nbr)
        def _():
            for f in range(min(_SLOTS, nbc)):
                chunk_copy(j, f).start()

        for f in range(nbc):
            @pl.when(f >= fj)
            def _(f=f):
                chunk_copy(j, f).wait()
                if f == nbc - 1 and widths[f] != bc:
                    src = last_ref[...]
                else:
                    src = stage_ref[f % _SLOTS]
                accb_ref[...] += jnp.dot(
                    src.astype(jnp.bfloat16),
                    s2row_ref[pl.ds(f * bc, widths[f]), :],
                    preferred_element_type=jnp.float32)
                if f + _SLOTS < nbc:
                    @pl.when(f + _SLOTS >= fj)
                    def _(f=f):
                        chunk_copy(j, f + _SLOTS).start()

        fj1 = ((j + 1) * br) // bc
        for f in range(nbc):
            @pl.when(jnp.logical_and(
                    t + 1 < 2 * nbr,
                    jnp.logical_and(f >= fj1, f < fj1 + _SLOTS)))
            def _(f=f):
                chunk_copy(j + 1, f).start()

        logits = accb_ref[...]
        m = jnp.max(logits, axis=1, keepdims=True)
        z = logits - m
        e = jnp.exp(z)
        s = jnp.sum(e, axis=1, keepdims=True)
        sm_ref[...] = e / s
        ls_ref[...] = z - jnp.log(s)


def kernel(x, adj, W1, b1, W2, b2):
    n, f_in = x.shape
    h_dim = W1.shape[1]
    c_dim = W2.shape[1]
    br = _pick_br(n)
    nbr = n // br
    bc = 1024 if n >= 4096 else 256
    nbc = -(-n // bc)

    b1r = b1.reshape(1, h_dim)
    b2r = b2.reshape(1, c_dim)

    ls, sm = pl.pallas_call(
        functools.partial(_gcn_kernel, n, br, nbr, bc, nbc, h_dim),
        grid=(2 * nbr,),
        in_specs=[
            pl.BlockSpec((n, f_in), lambda t: (0, 0)),
            pl.BlockSpec((br, n), lambda t: (jnp.minimum(t, nbr - 1), 0)),
            pl.BlockSpec(memory_space=pl.ANY),
            pl.BlockSpec((f_in, h_dim), lambda t: (0, 0)),
            pl.BlockSpec((1, h_dim), lambda t: (0, 0)),
            pl.BlockSpec((h_dim, c_dim), lambda t: (0, 0)),
            pl.BlockSpec((1, c_dim), lambda t: (0, 0)),
        ],
        out_specs=[
            pl.BlockSpec((br, c_dim), lambda t: (jnp.maximum(t - nbr, 0), 0)),
            pl.BlockSpec((br, c_dim), lambda t: (jnp.maximum(t - nbr, 0), 0)),
        ],
        out_shape=[
            jax.ShapeDtypeStruct((n, c_dim), jnp.float32),
            jax.ShapeDtypeStruct((n, c_dim), jnp.float32),
        ],
        scratch_shapes=[
            pltpu.VMEM((n, h_dim + c_dim), jnp.bfloat16),
            pltpu.VMEM((n, c_dim), jnp.bfloat16),
            pltpu.VMEM((n, c_dim), jnp.float32),
            pltpu.VMEM((br, c_dim), jnp.float32),
            pltpu.VMEM((_SLOTS, br, bc), jnp.float32),
            pltpu.VMEM((br, n - (nbc - 1) * bc), jnp.float32),
            pltpu.SemaphoreType.DMA((_SLOTS + 1,)),
        ],
    )(x, adj, adj, W1, b1r, W2, b2r)
    return ls, sm
